# baseline (speedup 1.0000x reference)
import jax
import jax.numpy as jnp
from jax import lax
from jax.experimental import pallas as pl
from jax.experimental.pallas import tpu as pltpu
from jax.experimental.pallas import tpu_sc as plsc

B = 1024
D = 1024
K = 8192
LR = 0.001
EPS = 1e-8
TK = 1024
KT = K // TK

_NC = 2
_NS = 16
_NW = _NC * _NS
_BPW = B // _NW
_LANES = 16


def _assign_body(dp_ref, c_ref, maxv_ref, idx_ref, ctr_ref):
    kt = pl.program_id(0)
    c = c_ref[...]
    dpb = dp_ref[...].astype(jnp.bfloat16)
    num = jax.lax.dot_general(
        dpb, c.astype(jnp.bfloat16), (((1,), (1,)), ((), ())),
        preferred_element_type=jnp.float32,
    )
    cn2 = jnp.sum(c * c, axis=1, keepdims=True)
    rs = jax.lax.rsqrt(jnp.maximum(cn2, 1e-35))
    scores = num * rs.reshape(1, TK)
    tmax = jnp.max(scores, axis=1, keepdims=True)
    col = jax.lax.broadcasted_iota(jnp.int32, (B, TK), 1)
    targ = jnp.min(
        jnp.where(scores == tmax, col, K), axis=1, keepdims=True
    ) + kt * TK

    @pl.when(kt == 0)
    def _():
        maxv_ref[...] = tmax
        idx_ref[...] = targ

    @pl.when(kt > 0)
    def _():
        better = tmax > maxv_ref[...]
        m = jnp.where(better, tmax, maxv_ref[...])
        t = jnp.where(better, targ, idx_ref[...])
        maxv_ref[...] = m
        idx_ref[...] = t

        @pl.when(kt == KT - 1)
        def _():
            idxf = t.astype(jnp.float32)
            eye = (
                jax.lax.broadcasted_iota(jnp.int32, (B, B), 0)
                == jax.lax.broadcasted_iota(jnp.int32, (B, B), 1)
            ).astype(jnp.float32)
            idxr = jax.lax.dot_general(
                idxf, eye, (((0,), (0,)), ((), ())),
                preferred_element_type=jnp.float32,
            )
            adj = (idxf == idxr).astype(jnp.bfloat16)
            rows = jax.lax.dot_general(
                adj, dpb, (((1,), (0,)), ((), ())),
                preferred_element_type=jnp.float32,
            )
            cnt = jax.lax.dot_general(
                adj, jnp.ones((B, 1), jnp.bfloat16), (((1,), (0,)), ((), ())),
                preferred_element_type=jnp.float32,
            )
            ctr_ref[...] = LR * (rows / cnt)


_RPW = K // _NW


_CH = 32
_NBUF = 3


def _sc_copy_body(cent_hbm, out_hbm, b0, b1, b2, l0, l1, l2, s0, s1, s2):
    wid = lax.axis_index("s") * _NC + lax.axis_index("c")
    base = wid * _RPW
    bufs = (b0, b1, b2)
    lsem = (l0, l1, l2)
    ssem = (s0, s1, s2)
    nch = _RPW // _CH
    ld = {}
    st = {}
    for t in range(min(_NBUF, nch)):
        ld[t] = pltpu.async_copy(
            cent_hbm.at[pl.ds(base + t * _CH, _CH)], bufs[t % _NBUF], lsem[t % _NBUF]
        )
    for t in range(nch):
        i = t % _NBUF
        ld[t].wait()
        st[t] = pltpu.async_copy(
            bufs[i], out_hbm.at[pl.ds(base + t * _CH, _CH)], ssem[i]
        )
        nt = t + _NBUF
        if nt < nch:
            st[t].wait()
            ld[nt] = pltpu.async_copy(
                cent_hbm.at[pl.ds(base + nt * _CH, _CH)], bufs[i], lsem[i]
            )
    for t in range(max(0, nch - _NBUF), nch):
        st[t].wait()


def _sc_update_body(idx_hbm, ctr_hbm, cent_hbm, out_hbm, idx_v, rows_v, ctr_v, sem):
    wid = lax.axis_index("s") * _NC + lax.axis_index("c")
    base = wid * _BPW
    pltpu.sync_copy(idx_hbm.at[pl.ds(base, _BPW)], idx_v)
    gather = pltpu.async_copy(cent_hbm.at[idx_v], rows_v, sem)
    pltpu.sync_copy(ctr_hbm.at[pl.ds(base, _BPW)], ctr_v)
    gather.wait()

    def body(i, _):
        for j in range(D // _LANES):
            sl = pl.ds(j * _LANES, _LANES)
            rows_v[i, sl] = rows_v[i, sl] * (1.0 - LR) + ctr_v[i, sl]
        return 0

    lax.fori_loop(0, _BPW, body, 0)
    pltpu.async_copy(rows_v, out_hbm.at[idx_v], sem).wait()


def kernel(datapoints, batch_cos_sim, centroid):
    del batch_cos_sim
    dp = datapoints
    sc_copy = pl.kernel(
        _sc_copy_body,
        out_type=jax.ShapeDtypeStruct((K, D), jnp.float32),
        mesh=plsc.VectorSubcoreMesh(core_axis_name="c", subcore_axis_name="s"),
        scratch_types=[
            pltpu.VMEM((_CH, D), jnp.float32),
            pltpu.VMEM((_CH, D), jnp.float32),
            pltpu.VMEM((_CH, D), jnp.float32),
            pltpu.SemaphoreType.DMA,
            pltpu.SemaphoreType.DMA,
            pltpu.SemaphoreType.DMA,
            pltpu.SemaphoreType.DMA,
            pltpu.SemaphoreType.DMA,
            pltpu.SemaphoreType.DMA,
        ],
    )
    out0 = sc_copy(centroid)

    _, idx, contrib = pl.pallas_call(
        _assign_body,
        grid=(KT,),
        in_specs=[
            pl.BlockSpec((B, D), lambda k: (0, 0)),
            pl.BlockSpec((TK, D), lambda k: (k, 0)),
        ],
        out_specs=[
            pl.BlockSpec((B, 1), lambda k: (0, 0)),
            pl.BlockSpec((B, 1), lambda k: (0, 0)),
            pl.BlockSpec((B, D), lambda k: (0, 0)),
        ],
        out_shape=[
            jax.ShapeDtypeStruct((B, 1), jnp.float32),
            jax.ShapeDtypeStruct((B, 1), jnp.int32),
            jax.ShapeDtypeStruct((B, D), jnp.float32),
        ],
    )(dp, centroid)

    sc_update = pl.kernel(
        _sc_update_body,
        out_type=(),
        mesh=plsc.VectorSubcoreMesh(core_axis_name="c", subcore_axis_name="s"),
        scratch_types=[
            pltpu.VMEM((_BPW,), jnp.int32),
            pltpu.VMEM((_BPW, D), jnp.float32),
            pltpu.VMEM((_BPW, D), jnp.float32),
            pltpu.SemaphoreType.DMA,
        ],
    )
    out_ref = jax.new_ref(out0)
    sc_update(idx.reshape(B), contrib, centroid, out_ref)
    return jax.freeze(out_ref)


# speedup vs baseline: 1.0052x; 1.0052x over previous
import jax
import jax.numpy as jnp
from jax import lax
from jax.experimental import pallas as pl
from jax.experimental.pallas import tpu as pltpu
from jax.experimental.pallas import tpu_sc as plsc

B = 1024
D = 1024
K = 8192
LR = 0.001
EPS = 1e-8
TK = 512
KT = K // TK

_NC = 2
_NS = 16
_NW = _NC * _NS
_BPW = B // _NW
_LANES = 16


def _assign_body(dp_ref, c_ref, maxv_ref, idx_ref, ctr_ref, out0_ref):
    kt = pl.program_id(0)
    c = c_ref[...]
    out0_ref[...] = c
    dpb = dp_ref[...].astype(jnp.bfloat16)
    num = jax.lax.dot_general(
        dpb, c.astype(jnp.bfloat16), (((1,), (1,)), ((), ())),
        preferred_element_type=jnp.float32,
    )
    cn2 = jnp.sum(c * c, axis=1, keepdims=True)
    rs = jax.lax.rsqrt(jnp.maximum(cn2, 1e-35))
    scores = num * rs.reshape(1, TK)
    tmax = jnp.max(scores, axis=1, keepdims=True)
    col = jax.lax.broadcasted_iota(jnp.int32, (B, TK), 1)
    targ = jnp.min(
        jnp.where(scores == tmax, col, K), axis=1, keepdims=True
    ) + kt * TK

    @pl.when(kt == 0)
    def _():
        maxv_ref[...] = tmax
        idx_ref[...] = targ

    @pl.when(kt > 0)
    def _():
        better = tmax > maxv_ref[...]
        m = jnp.where(better, tmax, maxv_ref[...])
        t = jnp.where(better, targ, idx_ref[...])
        maxv_ref[...] = m
        idx_ref[...] = t

        @pl.when(kt == KT - 1)
        def _():
            idxf = t.astype(jnp.float32)
            eye = (
                jax.lax.broadcasted_iota(jnp.int32, (B, B), 0)
                == jax.lax.broadcasted_iota(jnp.int32, (B, B), 1)
            ).astype(jnp.float32)
            idxr = jax.lax.dot_general(
                idxf, eye, (((0,), (0,)), ((), ())),
                preferred_element_type=jnp.float32,
            )
            adj = (idxf == idxr).astype(jnp.bfloat16)
            rows = jax.lax.dot_general(
                adj, dpb, (((1,), (0,)), ((), ())),
                preferred_element_type=jnp.float32,
            )
            cnt = jax.lax.dot_general(
                adj, jnp.ones((B, 1), jnp.bfloat16), (((1,), (0,)), ((), ())),
                preferred_element_type=jnp.float32,
            )
            ctr_ref[...] = LR * (rows / cnt)


def _sc_update_body(idx_hbm, ctr_hbm, cent_hbm, out_hbm, idx_v, rows_v, ctr_v, sem):
    wid = lax.axis_index("s") * _NC + lax.axis_index("c")
    base = wid * _BPW
    pltpu.sync_copy(idx_hbm.at[pl.ds(base, _BPW)], idx_v)
    gather = pltpu.async_copy(cent_hbm.at[idx_v], rows_v, sem)
    pltpu.sync_copy(ctr_hbm.at[pl.ds(base, _BPW)], ctr_v)
    gather.wait()

    def body(i, _):
        for j in range(D // _LANES):
            sl = pl.ds(j * _LANES, _LANES)
            rows_v[i, sl] = rows_v[i, sl] * (1.0 - LR) + ctr_v[i, sl]
        return 0

    lax.fori_loop(0, _BPW, body, 0)
    pltpu.async_copy(rows_v, out_hbm.at[idx_v], sem).wait()


def kernel(datapoints, batch_cos_sim, centroid):
    del batch_cos_sim
    dp = datapoints
    _, idx, contrib, out0 = pl.pallas_call(
        _assign_body,
        grid=(KT,),
        in_specs=[
            pl.BlockSpec((B, D), lambda k: (0, 0)),
            pl.BlockSpec((TK, D), lambda k: (k, 0)),
        ],
        out_specs=[
            pl.BlockSpec((B, 1), lambda k: (0, 0)),
            pl.BlockSpec((B, 1), lambda k: (0, 0)),
            pl.BlockSpec((B, D), lambda k: (0, 0)),
            pl.BlockSpec((TK, D), lambda k: (k, 0)),
        ],
        out_shape=[
            jax.ShapeDtypeStruct((B, 1), jnp.float32),
            jax.ShapeDtypeStruct((B, 1), jnp.int32),
            jax.ShapeDtypeStruct((B, D), jnp.float32),
            jax.ShapeDtypeStruct((K, D), jnp.float32),
        ],
    )(dp, centroid)

    sc_update = pl.kernel(
        _sc_update_body,
        out_type=(),
        mesh=plsc.VectorSubcoreMesh(core_axis_name="c", subcore_axis_name="s"),
        scratch_types=[
            pltpu.VMEM((_BPW,), jnp.int32),
            pltpu.VMEM((_BPW, D), jnp.float32),
            pltpu.VMEM((_BPW, D), jnp.float32),
            pltpu.SemaphoreType.DMA,
        ],
    )
    out_ref = jax.new_ref(out0)
    sc_update(idx.reshape(B), contrib, centroid, out_ref)
    return jax.freeze(out_ref)


# speedup vs baseline: 1.0924x; 1.0867x over previous
import jax
import jax.numpy as jnp
from jax import lax
from jax.experimental import pallas as pl
from jax.experimental.pallas import tpu as pltpu
from jax.experimental.pallas import tpu_sc as plsc

B = 1024
D = 1024
K = 8192
LR = 0.001
EPS = 1e-8
TK = 1024
KT = K // TK

_NC = 2
_NS = 16
_NW = _NC * _NS
_BPW = B // _NW
_LANES = 16


def _assign_body(dp_ref, c_ref, maxv_ref, idx_ref, ctr_ref, out0_ref):
    kt = pl.program_id(0)
    c = c_ref[...]
    out0_ref[...] = c
    dpb = dp_ref[...].astype(jnp.bfloat16)
    num = jax.lax.dot_general(
        dpb, c.astype(jnp.bfloat16), (((1,), (1,)), ((), ())),
        preferred_element_type=jnp.float32,
    )
    cn2 = jnp.sum(c * c, axis=1, keepdims=True)
    rs = jax.lax.rsqrt(jnp.maximum(cn2, 1e-35))
    scores = num * rs.reshape(1, TK)
    tmax = jnp.max(scores, axis=1, keepdims=True)
    col = jax.lax.broadcasted_iota(jnp.int32, (B, TK), 1)
    targ = jnp.min(
        jnp.where(scores == tmax, col, K), axis=1, keepdims=True
    ) + kt * TK

    @pl.when(kt == 0)
    def _():
        maxv_ref[...] = tmax
        idx_ref[...] = targ

    @pl.when(kt > 0)
    def _():
        better = tmax > maxv_ref[...]
        m = jnp.where(better, tmax, maxv_ref[...])
        t = jnp.where(better, targ, idx_ref[...])
        maxv_ref[...] = m
        idx_ref[...] = t

        @pl.when(kt == KT - 1)
        def _():
            idxf = t.astype(jnp.float32)
            idxr = idxf.reshape(1, B)
            adj = (idxf == idxr).astype(jnp.bfloat16)
            rows = jax.lax.dot_general(
                adj, dpb, (((1,), (0,)), ((), ())),
                preferred_element_type=jnp.float32,
            )
            cnt = jax.lax.dot_general(
                adj, jnp.ones((B, 1), jnp.bfloat16), (((1,), (0,)), ((), ())),
                preferred_element_type=jnp.float32,
            )
            ctr_ref[...] = LR * (rows / cnt)


def _sc_update_body(idx_hbm, ctr_hbm, cent_hbm, out_hbm, idx_v, rows_v, ctr_v, sem):
    wid = lax.axis_index("s") * _NC + lax.axis_index("c")
    base = wid * _BPW
    pltpu.sync_copy(idx_hbm.at[pl.ds(base, _BPW)], idx_v)
    gather = pltpu.async_copy(cent_hbm.at[idx_v], rows_v, sem)
    pltpu.sync_copy(ctr_hbm.at[pl.ds(base, _BPW)], ctr_v)
    gather.wait()

    def body(i, _):
        for j in range(D // _LANES):
            sl = pl.ds(j * _LANES, _LANES)
            rows_v[i, sl] = rows_v[i, sl] * (1.0 - LR) + ctr_v[i, sl]
        return 0

    lax.fori_loop(0, _BPW, body, 0)
    pltpu.async_copy(rows_v, out_hbm.at[idx_v], sem).wait()


def kernel(datapoints, batch_cos_sim, centroid):
    del batch_cos_sim
    dp = datapoints
    _, idx, contrib, out0 = pl.pallas_call(
        _assign_body,
        grid=(KT,),
        in_specs=[
            pl.BlockSpec((B, D), lambda k: (0, 0)),
            pl.BlockSpec((TK, D), lambda k: (k, 0)),
        ],
        out_specs=[
            pl.BlockSpec((B, 1), lambda k: (0, 0)),
            pl.BlockSpec((B, 1), lambda k: (0, 0)),
            pl.BlockSpec((B, D), lambda k: (0, 0)),
            pl.BlockSpec((TK, D), lambda k: (k, 0)),
        ],
        out_shape=[
            jax.ShapeDtypeStruct((B, 1), jnp.float32),
            jax.ShapeDtypeStruct((B, 1), jnp.int32),
            jax.ShapeDtypeStruct((B, D), jnp.float32),
            jax.ShapeDtypeStruct((K, D), jnp.float32),
        ],
    )(dp, centroid)

    sc_update = pl.kernel(
        _sc_update_body,
        out_type=(),
        mesh=plsc.VectorSubcoreMesh(core_axis_name="c", subcore_axis_name="s"),
        scratch_types=[
            pltpu.VMEM((_BPW,), jnp.int32),
            pltpu.VMEM((_BPW, D), jnp.float32),
            pltpu.VMEM((_BPW, D), jnp.float32),
            pltpu.SemaphoreType.DMA,
        ],
    )
    out_ref = jax.new_ref(out0)
    sc_update(idx.reshape(B), contrib, centroid, out_ref)
    return jax.freeze(out_ref)


# speedup vs baseline: 1.0954x; 1.0027x over previous
import jax
import jax.numpy as jnp
from jax import lax
from jax.experimental import pallas as pl
from jax.experimental.pallas import tpu as pltpu
from jax.experimental.pallas import tpu_sc as plsc

B = 1024
D = 1024
K = 8192
LR = 0.001
EPS = 1e-8
TK = 1024
KT = K // TK

_NC = 2
_NS = 16
_NW = _NC * _NS
_BPW = B // _NW
_LANES = 16


def _assign_body(dp_ref, c_ref, maxv_ref, idx_ref, ctr_ref, out0_ref):
    kt = pl.program_id(0)
    c = c_ref[...]
    out0_ref[...] = c
    dpb = dp_ref[...].astype(jnp.bfloat16)
    num = jax.lax.dot_general(
        dpb, c.astype(jnp.bfloat16), (((1,), (1,)), ((), ())),
        preferred_element_type=jnp.float32,
    )
    cn2 = jnp.sum(c * c, axis=1, keepdims=True)
    rs = jax.lax.rsqrt(jnp.maximum(cn2, 1e-35))
    scores = num * rs.reshape(1, TK)
    tmax = jnp.max(scores, axis=1, keepdims=True)
    col = jax.lax.broadcasted_iota(jnp.int32, (B, TK), 1)
    targ = jnp.min(
        jnp.where(scores == tmax, col, K), axis=1, keepdims=True
    ) + kt * TK

    @pl.when(kt == 0)
    def _():
        maxv_ref[...] = tmax
        idx_ref[...] = targ

    @pl.when(kt > 0)
    def _():
        better = tmax > maxv_ref[...]
        m = jnp.where(better, tmax, maxv_ref[...])
        t = jnp.where(better, targ, idx_ref[...])
        maxv_ref[...] = m
        idx_ref[...] = t

        @pl.when(kt == KT - 1)
        def _():
            idxf = t.astype(jnp.float32)
            idxr = idxf.reshape(1, B)
            adj = (idxf == idxr).astype(jnp.bfloat16)
            rows = jax.lax.dot_general(
                adj, dpb, (((1,), (0,)), ((), ())),
                preferred_element_type=jnp.float32,
            )
            cnt = jax.lax.dot_general(
                adj, jnp.ones((B, 1), jnp.bfloat16), (((1,), (0,)), ((), ())),
                preferred_element_type=jnp.float32,
            )
            ctr_ref[...] = LR * (rows / cnt)


_UCH = 8
_UNCH = _BPW // _UCH


def _sc_update_body(idx2_hbm, ctr_hbm, cent_hbm, out_hbm,
                    idx_v, g0, g1, c0, c1, sg0, sg1, sc0, sc1, so0, so1):
    wid = lax.axis_index("s") * _NC + lax.axis_index("c")
    rbase = wid * _UNCH
    pltpu.sync_copy(idx2_hbm.at[pl.ds(rbase, _UNCH)], idx_v)
    gb = (g0, g1)
    cb = (c0, c1)
    gs = (sg0, sg1)
    cs = (sc0, sc1)
    osm = (so0, so1)
    gld = {}
    cld = {}
    ost = {}

    def start(j):
        i = j % 2
        gld[j] = pltpu.async_copy(cent_hbm.at[idx_v.at[j]], gb[i], gs[i])
        cld[j] = pltpu.async_copy(
            ctr_hbm.at[pl.ds((rbase + j) * _UCH, _UCH)], cb[i], cs[i]
        )

    start(0)
    for j in range(_UNCH):
        i = j % 2
        if j + 1 < _UNCH:
            if j - 1 >= 0:
                ost[j - 1].wait()
            start(j + 1)
        gld[j].wait()
        cld[j].wait()

        def body(r, _):
            for q in range(D // _LANES):
                sl = pl.ds(q * _LANES, _LANES)
                gb[i][r, sl] = gb[i][r, sl] * (1.0 - LR) + cb[i][r, sl]
            return 0

        lax.fori_loop(0, _UCH, body, 0)
        ost[j] = pltpu.async_copy(gb[i], out_hbm.at[idx_v.at[j]], osm[i])
    ost[_UNCH - 1].wait()
    if _UNCH >= 2:
        ost[_UNCH - 2].wait()


def kernel(datapoints, batch_cos_sim, centroid):
    del batch_cos_sim
    dp = datapoints
    _, idx, contrib, out0 = pl.pallas_call(
        _assign_body,
        grid=(KT,),
        in_specs=[
            pl.BlockSpec((B, D), lambda k: (0, 0)),
            pl.BlockSpec((TK, D), lambda k: (k, 0)),
        ],
        out_specs=[
            pl.BlockSpec((B, 1), lambda k: (0, 0)),
            pl.BlockSpec((B, 1), lambda k: (0, 0)),
            pl.BlockSpec((B, D), lambda k: (0, 0)),
            pl.BlockSpec((TK, D), lambda k: (k, 0)),
        ],
        out_shape=[
            jax.ShapeDtypeStruct((B, 1), jnp.float32),
            jax.ShapeDtypeStruct((B, 1), jnp.int32),
            jax.ShapeDtypeStruct((B, D), jnp.float32),
            jax.ShapeDtypeStruct((K, D), jnp.float32),
        ],
    )(dp, centroid)

    sc_update = pl.kernel(
        _sc_update_body,
        out_type=(),
        mesh=plsc.VectorSubcoreMesh(core_axis_name="c", subcore_axis_name="s"),
        scratch_types=[
            pltpu.VMEM((_UNCH, _UCH), jnp.int32),
            pltpu.VMEM((_UCH, D), jnp.float32),
            pltpu.VMEM((_UCH, D), jnp.float32),
            pltpu.VMEM((_UCH, D), jnp.float32),
            pltpu.VMEM((_UCH, D), jnp.float32),
            pltpu.SemaphoreType.DMA,
            pltpu.SemaphoreType.DMA,
            pltpu.SemaphoreType.DMA,
            pltpu.SemaphoreType.DMA,
            pltpu.SemaphoreType.DMA,
            pltpu.SemaphoreType.DMA,
        ],
    )
    out_ref = jax.new_ref(out0)
    sc_update(idx.reshape(B // _UCH, _UCH), contrib, centroid, out_ref)
    return jax.freeze(out_ref)
